# baseline (speedup 1.0000x reference)
import functools

import jax
import jax.numpy as jnp
from jax import lax
from jax.experimental import pallas as pl
from jax.experimental.pallas import tpu as pltpu
from jax.experimental.pallas import tpu_sc as plsc

D_MODEL = 4096
N_EXP = 64
TOP_K = 8
TOKENS = 32768

NC = 2
NS = 16
NW = NC * NS
LANES = 16

CHUNKS = (12288, 12288, 8192)

BT = 512


def _matmul_body(x_ref, w_ref, o_ref):
    o_ref[...] = lax.dot_general(
        x_ref[...], w_ref[...],
        dimension_numbers=(((1,), (1,)), ((), ())),
        preferred_element_type=jnp.float32,
    )


def _logits_tc(x, W, off, ch):
    return pl.pallas_call(
        _matmul_body,
        grid=(ch // BT,),
        in_specs=[
            pl.BlockSpec((BT, D_MODEL),
                         lambda i, off=off: (off // BT + i, 0)),
            pl.BlockSpec((N_EXP, D_MODEL), lambda i: (0, 0)),
        ],
        out_specs=pl.BlockSpec((BT, N_EXP), lambda i: (i, 0)),
        out_shape=jax.ShapeDtypeStruct((ch, N_EXP), jnp.float32),
    )(x, W)


_mesh = plsc.VectorSubcoreMesh(
    core_axis_name="c", subcore_axis_name="s", num_cores=NC, num_subcores=NS)


@functools.lru_cache(maxsize=None)
def _make_topk_sc(ch):
    tpw = ch // NW

    @functools.partial(
        pl.kernel,
        mesh=_mesh,
        out_type=[
            jax.ShapeDtypeStruct((ch * TOP_K,), jnp.float32),
            jax.ShapeDtypeStruct((ch * TOP_K,), jnp.int32),
        ],
        scratch_types=[
            pltpu.VMEM((tpw, N_EXP), jnp.float32),
            pltpu.VMEM((tpw * TOP_K + LANES - TOP_K,), jnp.float32),
            pltpu.VMEM((tpw * TOP_K + LANES - TOP_K,), jnp.int32),
        ],
        compiler_params=pltpu.CompilerParams(
            needs_layout_passes=False, use_tc_tiling_on_sc=False),
    )
    def _topk_sc(logits_hbm, probs_hbm, idx_hbm, lv, pv, iv):
        wid = lax.axis_index("s") * NC + lax.axis_index("c")
        base = wid * tpw
        pltpu.sync_copy(logits_hbm.at[pl.ds(base, tpw), :], lv)

        lane = lax.iota(jnp.int32, LANES)
        lo_mask = lane < TOP_K

        def merge(va, ia, vb, ib):
            vbr = lax.rev(vb, (0,))
            ibr = lax.rev(ib, (0,))
            vm = jnp.where(lo_mask, va, vbr)
            im = jnp.where(lo_mask, ia, ibr)
            return plsc.sort_key_val(vm, im, descending=True)

        @plsc.parallel_loop(0, tpw, unroll=4)
        def body(t):
            sv = []
            si = []
            for g in range(N_EXP // LANES):
                v = lv[t, pl.ds(g * LANES, LANES)]
                s_v, s_i = plsc.sort_key_val(
                    v, lane + g * LANES, descending=True)
                sv.append(s_v)
                si.append(s_i)
            v01, i01 = merge(sv[0], si[0], sv[1], si[1])
            v23, i23 = merge(sv[2], si[2], sv[3], si[3])
            v, i = merge(v01, i01, v23, i23)

            m = lax.reduce_max(v, axes=(0,))
            e = jnp.where(lo_mask, jnp.exp(v - m), 0.0)
            s = lax.reduce_sum(e, axes=(0,))
            p = e / s

            plsc.store_compressed(
                pv.at[pl.ds(t * TOP_K, LANES)], p, mask=lo_mask)
            plsc.store_compressed(
                iv.at[pl.ds(t * TOP_K, LANES)], i, mask=lo_mask)

        pltpu.sync_copy(pv.at[pl.ds(0, tpw * TOP_K)],
                        probs_hbm.at[pl.ds(base * TOP_K, tpw * TOP_K)])
        pltpu.sync_copy(iv.at[pl.ds(0, tpw * TOP_K)],
                        idx_hbm.at[pl.ds(base * TOP_K, tpw * TOP_K)])

    return _topk_sc


def kernel(x, W):
    probs = []
    idxs = []
    off = 0
    for ch in CHUNKS:
        logits_c = _logits_tc(x, W, off, ch)
        p_c, i_c = _make_topk_sc(ch)(logits_c)
        probs.append(p_c.reshape(ch, TOP_K))
        idxs.append(i_c.reshape(ch, TOP_K))
        off += ch
    return (jnp.concatenate(probs, axis=0), jnp.concatenate(idxs, axis=0))


# speedup vs baseline: 1.0617x; 1.0617x over previous
import functools

import jax
import jax.numpy as jnp
from jax import lax
from jax.experimental import pallas as pl
from jax.experimental.pallas import tpu as pltpu
from jax.experimental.pallas import tpu_sc as plsc

D_MODEL = 4096
N_EXP = 64
TOP_K = 8
TOKENS = 32768

NC = 2
NS = 16
NW = NC * NS
LANES = 16

CHUNKS = (24576, 8192)

BT = 512


def _matmul_body(x_ref, w_ref, o_ref):
    o_ref[...] = lax.dot_general(
        x_ref[...], w_ref[...],
        dimension_numbers=(((1,), (1,)), ((), ())),
        preferred_element_type=jnp.float32,
    )


def _logits_tc(x, W, off, ch):
    return pl.pallas_call(
        _matmul_body,
        grid=(ch // BT,),
        in_specs=[
            pl.BlockSpec((BT, D_MODEL),
                         lambda i, off=off: (off // BT + i, 0)),
            pl.BlockSpec((N_EXP, D_MODEL), lambda i: (0, 0)),
        ],
        out_specs=pl.BlockSpec((BT, N_EXP), lambda i: (i, 0)),
        out_shape=jax.ShapeDtypeStruct((ch, N_EXP), jnp.float32),
    )(x, W)


_mesh = plsc.VectorSubcoreMesh(
    core_axis_name="c", subcore_axis_name="s", num_cores=NC, num_subcores=NS)


@functools.lru_cache(maxsize=None)
def _make_topk_sc(ch):
    tpw = ch // NW

    @functools.partial(
        pl.kernel,
        mesh=_mesh,
        out_type=[
            jax.ShapeDtypeStruct((ch * TOP_K,), jnp.float32),
            jax.ShapeDtypeStruct((ch * TOP_K,), jnp.int32),
        ],
        scratch_types=[
            pltpu.VMEM((tpw, N_EXP), jnp.float32),
            pltpu.VMEM((tpw * TOP_K + LANES - TOP_K,), jnp.float32),
            pltpu.VMEM((tpw * TOP_K + LANES - TOP_K,), jnp.int32),
        ],
        compiler_params=pltpu.CompilerParams(
            needs_layout_passes=False, use_tc_tiling_on_sc=False),
    )
    def _topk_sc(logits_hbm, probs_hbm, idx_hbm, lv, pv, iv):
        wid = lax.axis_index("s") * NC + lax.axis_index("c")
        base = wid * tpw
        pltpu.sync_copy(logits_hbm.at[pl.ds(base, tpw), :], lv)

        lane = lax.iota(jnp.int32, LANES)
        lo_mask = lane < TOP_K

        def merge(va, ia, vb, ib):
            vbr = lax.rev(vb, (0,))
            ibr = lax.rev(ib, (0,))
            vm = jnp.where(lo_mask, va, vbr)
            im = jnp.where(lo_mask, ia, ibr)
            return plsc.sort_key_val(vm, im, descending=True)

        @plsc.parallel_loop(0, tpw, unroll=4)
        def body(t):
            sv = []
            si = []
            for g in range(N_EXP // LANES):
                v = lv[t, pl.ds(g * LANES, LANES)]
                s_v, s_i = plsc.sort_key_val(
                    v, lane + g * LANES, descending=True)
                sv.append(s_v)
                si.append(s_i)
            v01, i01 = merge(sv[0], si[0], sv[1], si[1])
            v23, i23 = merge(sv[2], si[2], sv[3], si[3])
            v, i = merge(v01, i01, v23, i23)

            m = lax.reduce_max(v, axes=(0,))
            e = jnp.where(lo_mask, jnp.exp(v - m), 0.0)
            s = lax.reduce_sum(e, axes=(0,))
            p = e / s

            plsc.store_compressed(
                pv.at[pl.ds(t * TOP_K, LANES)], p, mask=lo_mask)
            plsc.store_compressed(
                iv.at[pl.ds(t * TOP_K, LANES)], i, mask=lo_mask)

        pltpu.sync_copy(pv.at[pl.ds(0, tpw * TOP_K)],
                        probs_hbm.at[pl.ds(base * TOP_K, tpw * TOP_K)])
        pltpu.sync_copy(iv.at[pl.ds(0, tpw * TOP_K)],
                        idx_hbm.at[pl.ds(base * TOP_K, tpw * TOP_K)])

    return _topk_sc


def kernel(x, W):
    probs = []
    idxs = []
    off = 0
    for ch in CHUNKS:
        logits_c = _logits_tc(x, W, off, ch)
        p_c, i_c = _make_topk_sc(ch)(logits_c)
        probs.append(p_c.reshape(ch, TOP_K))
        idxs.append(i_c.reshape(ch, TOP_K))
        off += ch
    return (jnp.concatenate(probs, axis=0), jnp.concatenate(idxs, axis=0))


# speedup vs baseline: 1.1058x; 1.0415x over previous
import functools

import jax
import jax.numpy as jnp
from jax import lax
from jax.experimental import pallas as pl
from jax.experimental.pallas import tpu as pltpu
from jax.experimental.pallas import tpu_sc as plsc

D_MODEL = 4096
N_EXP = 64
TOP_K = 8
TOKENS = 32768

NC = 2
NS = 16
NW = NC * NS
LANES = 16

CHUNKS = (16384, 16384)

BT = 512
HB = BT // 2


def _matmul_body(x_ref, w_ref, o_ref):
    logits = lax.dot_general(
        x_ref[...], w_ref[...],
        dimension_numbers=(((1,), (1,)), ((), ())),
        preferred_element_type=jnp.float32,
    )
    o_ref[...] = jnp.concatenate([logits[:HB], logits[HB:]], axis=1)


def _logits_tc(x, W, off, ch):
    return pl.pallas_call(
        _matmul_body,
        grid=(ch // BT,),
        in_specs=[
            pl.BlockSpec((BT, D_MODEL),
                         lambda i, off=off: (off // BT + i, 0)),
            pl.BlockSpec((N_EXP, D_MODEL), lambda i: (0, 0)),
        ],
        out_specs=pl.BlockSpec((BT // 2, 2 * N_EXP), lambda i: (i, 0)),
        out_shape=jax.ShapeDtypeStruct((ch // 2, 2 * N_EXP), jnp.float32),
    )(x, W)


_mesh = plsc.VectorSubcoreMesh(
    core_axis_name="c", subcore_axis_name="s", num_cores=NC, num_subcores=NS)


@functools.lru_cache(maxsize=None)
def _make_topk_sc(ch):
    tpw = ch // NW
    rpw = tpw // 2
    assert tpw % BT == 0

    @functools.partial(
        pl.kernel,
        mesh=_mesh,
        out_type=[
            jax.ShapeDtypeStruct((ch * TOP_K,), jnp.float32),
            jax.ShapeDtypeStruct((ch * TOP_K,), jnp.int32),
        ],
        scratch_types=[
            pltpu.VMEM((rpw, 2 * N_EXP), jnp.float32),
            pltpu.VMEM((tpw * TOP_K + LANES - TOP_K,), jnp.float32),
            pltpu.VMEM((tpw * TOP_K + LANES - TOP_K,), jnp.int32),
        ],
        compiler_params=pltpu.CompilerParams(
            needs_layout_passes=False, use_tc_tiling_on_sc=False),
    )
    def _topk_sc(logits_hbm, probs_hbm, idx_hbm, lv, pv, iv):
        wid = lax.axis_index("s") * NC + lax.axis_index("c")
        base = wid * tpw
        pltpu.sync_copy(logits_hbm.at[pl.ds(wid * rpw, rpw), :], lv)

        lane = lax.iota(jnp.int32, LANES)
        lo_mask = lane < TOP_K

        def merge(va, ia, vb, ib):
            vbr = lax.rev(vb, (0,))
            ibr = lax.rev(ib, (0,))
            vm = jnp.where(lo_mask, va, vbr)
            im = jnp.where(lo_mask, ia, ibr)
            return plsc.sort_key_val(vm, im, descending=True)

        def one_token(r, h):
            sv = []
            si = []
            for g in range(N_EXP // LANES):
                v = lv[r, pl.ds(h * N_EXP + g * LANES, LANES)]
                s_v, s_i = plsc.sort_key_val(
                    v, lane + g * LANES, descending=True)
                sv.append(s_v)
                si.append(s_i)
            v01, i01 = merge(sv[0], si[0], sv[1], si[1])
            v23, i23 = merge(sv[2], si[2], sv[3], si[3])
            v, i = merge(v01, i01, v23, i23)

            m = lax.reduce_max(v, axes=(0,))
            e = jnp.where(lo_mask, jnp.exp(v - m), 0.0)
            s = lax.reduce_sum(e, axes=(0,))
            p = e / s

            t = (r // HB) * BT + h * HB + lax.rem(r, HB)
            plsc.store_compressed(
                pv.at[pl.ds(t * TOP_K, LANES)], p, mask=lo_mask)
            plsc.store_compressed(
                iv.at[pl.ds(t * TOP_K, LANES)], i, mask=lo_mask)

        @plsc.parallel_loop(0, rpw, unroll=2)
        def body(r):
            one_token(r, 0)
            one_token(r, 1)

        pltpu.sync_copy(pv.at[pl.ds(0, tpw * TOP_K)],
                        probs_hbm.at[pl.ds(base * TOP_K, tpw * TOP_K)])
        pltpu.sync_copy(iv.at[pl.ds(0, tpw * TOP_K)],
                        idx_hbm.at[pl.ds(base * TOP_K, tpw * TOP_K)])

    return _topk_sc


def kernel(x, W):
    probs = []
    idxs = []
    off = 0
    for ch in CHUNKS:
        logits_c = _logits_tc(x, W, off, ch)
        p_c, i_c = _make_topk_sc(ch)(logits_c)
        probs.append(p_c)
        idxs.append(i_c)
        off += ch
    return (jnp.concatenate(probs).reshape(TOKENS, TOP_K),
            jnp.concatenate(idxs).reshape(TOKENS, TOP_K))
